# Initial kernel scaffold; baseline (speedup 1.0000x reference)
import jax
import jax.numpy as jnp
from jax.experimental import pallas as pl


def kernel(users, items, user_table, item_table):
    raise NotImplementedError("write your pallas kernel here")


# speedup vs baseline: 1.1962x; 1.1962x over previous
import functools

import jax
import jax.numpy as jnp
from jax import lax
from jax.experimental import pallas as pl
from jax.experimental.pallas import tpu as pltpu
from jax.experimental.pallas import tpu_sc as plsc

NC = 2
NS = 16
L = 16
NW = NC * NS

B = 16384
D = 128
CHUNK = 128
PER_W = B // NW
NCHUNK = PER_W // CHUNK


def _gmf_body(users_hbm, items_hbm, utab_hbm, itab_hbm, out_hbm,
              idx_u, idx_i, rows_u, rows_i, sem_u, sem_i):
    wid = lax.axis_index("s") * NC + lax.axis_index("c")
    base_w = wid * PER_W
    for c in range(NCHUNK):
        base = base_w + c * CHUNK
        pltpu.sync_copy(users_hbm.at[pl.ds(base, CHUNK)], idx_u)
        pltpu.sync_copy(items_hbm.at[pl.ds(base, CHUNK)], idx_i)
        cu = pltpu.async_copy(utab_hbm.at[idx_u], rows_u, sem_u)
        ci = pltpu.async_copy(itab_hbm.at[idx_i], rows_i, sem_i)
        cu.wait()
        ci.wait()

        def mul_row(r, carry):
            for j in range(D // L):
                sl = pl.ds(j * L, L)
                rows_u[r, sl] = rows_u[r, sl] * rows_i[r, sl]
            return carry

        lax.fori_loop(0, CHUNK, mul_row, 0)
        pltpu.sync_copy(rows_u, out_hbm.at[pl.ds(base, CHUNK)])


_gmf = functools.partial(
    pl.kernel,
    out_type=jax.ShapeDtypeStruct((B, D), jnp.float32),
    mesh=plsc.VectorSubcoreMesh(
        core_axis_name="c", subcore_axis_name="s",
        num_cores=NC, num_subcores=NS),
    scratch_types=[
        pltpu.VMEM((CHUNK,), jnp.int32),
        pltpu.VMEM((CHUNK,), jnp.int32),
        pltpu.VMEM((CHUNK, D), jnp.float32),
        pltpu.VMEM((CHUNK, D), jnp.float32),
        pltpu.SemaphoreType.DMA,
        pltpu.SemaphoreType.DMA,
    ],
)(_gmf_body)


def kernel(users, items, user_table, item_table):
    return _gmf(users.astype(jnp.int32), items.astype(jnp.int32),
                user_table, item_table)


# speedup vs baseline: 1.4686x; 1.2277x over previous
import functools

import jax
import jax.numpy as jnp
from jax import lax
from jax.experimental import pallas as pl
from jax.experimental.pallas import tpu as pltpu
from jax.experimental.pallas import tpu_sc as plsc

NC = 2
NS = 16
L = 16
NW = NC * NS

B = 16384
D = 128
CHUNK = 128
PER_W = B // NW
NCHUNK = PER_W // CHUNK


def _gmf_body(users_hbm, items_hbm, utab_hbm, itab_hbm, out_hbm,
              idx_u, idx_i, ru0, ri0, ru1, ri1,
              sem_g0, sem_g1, sem_o0, sem_o1):
    wid = lax.axis_index("s") * NC + lax.axis_index("c")
    base_w = wid * PER_W
    pltpu.sync_copy(users_hbm.at[pl.ds(base_w, PER_W)], idx_u)
    pltpu.sync_copy(items_hbm.at[pl.ds(base_w, PER_W)], idx_i)

    ru = [ru0, ru1]
    ri = [ri0, ri1]
    sem_g = [sem_g0, sem_g1]
    sem_o = [sem_o0, sem_o1]

    def start_gathers(c):
        b = c % 2
        s = pl.ds(c * CHUNK, CHUNK)
        cu = pltpu.async_copy(utab_hbm.at[idx_u.at[s]], ru[b], sem_g[b])
        ci = pltpu.async_copy(itab_hbm.at[idx_i.at[s]], ri[b], sem_g[b])
        return cu, ci

    gathers = {0: start_gathers(0)}
    out_copies = {}
    for c in range(NCHUNK):
        b = c % 2
        if c + 1 < NCHUNK:
            if c - 1 in out_copies:
                out_copies[c - 1].wait()
            gathers[c + 1] = start_gathers(c + 1)
        cu, ci = gathers[c]
        cu.wait()
        ci.wait()

        def mul_row(r, carry):
            for j in range(D // L):
                sl = pl.ds(j * L, L)
                ru[b][r, sl] = ru[b][r, sl] * ri[b][r, sl]
            return carry

        lax.fori_loop(0, CHUNK, mul_row, 0)
        out_copies[c] = pltpu.async_copy(
            ru[b], out_hbm.at[pl.ds(base_w + c * CHUNK, CHUNK)], sem_o[b])
    out_copies[NCHUNK - 2].wait()
    out_copies[NCHUNK - 1].wait()


_gmf = functools.partial(
    pl.kernel,
    out_type=jax.ShapeDtypeStruct((B, D), jnp.float32),
    mesh=plsc.VectorSubcoreMesh(
        core_axis_name="c", subcore_axis_name="s",
        num_cores=NC, num_subcores=NS),
    scratch_types=[
        pltpu.VMEM((PER_W,), jnp.int32),
        pltpu.VMEM((PER_W,), jnp.int32),
        pltpu.VMEM((CHUNK, D), jnp.float32),
        pltpu.VMEM((CHUNK, D), jnp.float32),
        pltpu.VMEM((CHUNK, D), jnp.float32),
        pltpu.VMEM((CHUNK, D), jnp.float32),
        pltpu.SemaphoreType.DMA,
        pltpu.SemaphoreType.DMA,
        pltpu.SemaphoreType.DMA,
        pltpu.SemaphoreType.DMA,
    ],
)(_gmf_body)


def kernel(users, items, user_table, item_table):
    return _gmf(users.astype(jnp.int32), items.astype(jnp.int32),
                user_table, item_table)
